# Initial kernel scaffold; baseline (speedup 1.0000x reference)
import jax
import jax.numpy as jnp
from jax.experimental import pallas as pl


def kernel(x, edge_index, W_sd, b_sd, W_ds, b_ds, W0, b0, W1, b1, W2, b2, W3, b3, alpha, beta, gama):
    raise NotImplementedError("write your pallas kernel here")


# speedup vs baseline: 7.7094x; 7.7094x over previous
import functools

import jax
import jax.numpy as jnp
from jax import lax
from jax.experimental import pallas as pl
from jax.experimental.pallas import tpu as pltpu
from jax.experimental.pallas import tpu_sc as plsc

N = 10000
NPAD = 10240
NC, NS = 2, 16
NW = NC * NS
K = 128
NB = 79
EPT = NB * K
EPAD = NW * EPT
JUNK = NPAD - 1
ROWS_PER_TILE = NPAD // NS


def _make_spmm(D):
    mesh = plsc.VectorSubcoreMesh(core_axis_name="c", subcore_axis_name="s")

    @functools.partial(
        pl.kernel,
        out_type=jax.ShapeDtypeStruct((NC, NPAD, D), jnp.float32),
        mesh=mesh,
        scratch_types=[
            pltpu.VMEM((NB, K), jnp.int32),
            pltpu.VMEM((NB, K), jnp.int32),
            pltpu.VMEM((K, D), jnp.float32),
            pltpu.VMEM_SHARED((NPAD, D), jnp.float32),
            pltpu.SemaphoreType.DMA,
        ],
        compiler_params=pltpu.CompilerParams(use_tc_tiling_on_sc=False),
    )
    def spmm(dst_hbm, src_hbm, z_hbm, out_hbm, idx_dst, idx_src, rows, acc, gsem):
        c = lax.axis_index("c")
        s = lax.axis_index("s")
        w = c * NS + s

        pltpu.sync_copy(dst_hbm.at[w], idx_dst)
        pltpu.sync_copy(src_hbm.at[w], idx_src)

        def zrow(i, carry):
            for j in range(D // 16):
                rows[i, pl.ds(j * 16, 16)] = jnp.zeros((16,), jnp.float32)
            return carry

        lax.fori_loop(0, K, zrow, 0)

        def zacc(j, carry):
            pltpu.sync_copy(rows, acc.at[pl.ds(s * ROWS_PER_TILE + j * K, K)])
            return carry

        lax.fori_loop(0, ROWS_PER_TILE // K, zacc, 0)
        plsc.subcore_barrier()

        def step(b, carry):
            pltpu.async_copy(z_hbm.at[idx_src.at[b]], rows, gsem).wait()
            pltpu.sync_copy(rows, acc.at[idx_dst.at[b]], add=True)
            return carry

        lax.fori_loop(0, NB, step, 0)
        plsc.subcore_barrier()

        pltpu.sync_copy(
            acc.at[pl.ds(s * ROWS_PER_TILE, ROWS_PER_TILE)],
            out_hbm.at[c, pl.ds(s * ROWS_PER_TILE, ROWS_PER_TILE)],
        )

    return spmm


_spmm16 = _make_spmm(16)
_spmm128 = _make_spmm(128)


def _tc_combine(hcat, wcat, bias):
    BN = 512

    def body(h_ref, w_ref, b_ref, o_ref):
        o_ref[...] = (
            jnp.dot(h_ref[...], w_ref[...], preferred_element_type=jnp.float32)
            + b_ref[...]
        )

    return pl.pallas_call(
        body,
        grid=(NPAD // BN,),
        in_specs=[
            pl.BlockSpec((BN, 768), lambda i: (i, 0)),
            pl.BlockSpec((768, 128), lambda i: (0, 0)),
            pl.BlockSpec((1, 128), lambda i: (0, 0)),
        ],
        out_specs=pl.BlockSpec((BN, 128), lambda i: (i, 0)),
        out_shape=jax.ShapeDtypeStruct((NPAD, 128), jnp.float32),
    )(hcat, wcat, bias)


def _inv_sqrt(d):
    return jnp.where(d > 0, 1.0 / jnp.sqrt(jnp.where(d > 0, d, 1.0)), 0.0)


def _col16(*cols):
    z = [c[:, None] for c in cols]
    z.append(jnp.zeros((N, 16 - len(cols)), jnp.float32))
    return jnp.concatenate(z, axis=1)


def kernel(x, edge_index, W_sd, b_sd, W_ds, b_ds, W0, b0, W1, b1, W2, b2,
           W3, b3, alpha, beta, gama):
    row, col = edge_index[0], edge_index[1]
    pad = EPAD - row.shape[0]
    junk = jnp.full((pad,), JUNK, jnp.int32)
    zero = jnp.zeros((pad,), jnp.int32)
    dstS = jnp.concatenate([row, junk]).reshape(NW, NB, K)
    srcS = jnp.concatenate([col, zero]).reshape(NW, NB, K)
    dstT = jnp.concatenate([col, junk]).reshape(NW, NB, K)
    srcT = jnp.concatenate([row, zero]).reshape(NW, NB, K)

    def S16(z):
        p = _spmm16(dstS, srcS, z)
        return (p[0] + p[1])[:N]

    def T16(z):
        p = _spmm16(dstT, srcT, z)
        return (p[0] + p[1])[:N]

    def S128(z):
        p = _spmm128(dstS, srcS, z)
        return (p[0] + p[1])[:N]

    def T128(z):
        p = _spmm128(dstT, srcT, z)
        return (p[0] + p[1])[:N]

    ones16 = jnp.ones((N, 16), jnp.float32)
    out_deg = S16(ones16)[:, 0]
    in_deg = T16(ones16)[:, 0]
    dout = _inv_sqrt(out_deg)
    din = _inv_sqrt(in_deg)

    q = dout * S16(_col16(din))[:, 0]
    p = din * T16(_col16(dout))[:, 0]

    r13 = S16(_col16(din * p, din * q))
    r1 = dout * r13[:, 0]
    r3 = dout * r13[:, 1]
    r24 = T16(_col16(dout * q, dout * p))
    r2 = din * r24[:, 0]
    r4 = din * r24[:, 1]
    c1, c2, c3, c4 = _inv_sqrt(r1), _inv_sqrt(r2), _inv_sqrt(r3), _inv_sqrt(r4)

    U1 = S128(din[:, None] * x)
    V2 = S128((din * c2)[:, None] * x)
    V3 = S128((din * c4)[:, None] * x)
    U2 = T128(dout[:, None] * x)
    V1 = T128((dout * c1)[:, None] * x)
    V4 = T128((dout * c3)[:, None] * x)

    H3c = S128((din * din)[:, None] * V1)
    H5c = S128((din * dout)[:, None] * V3)
    H4c = T128((dout * dout)[:, None] * V2)
    H6c = T128((dout * din)[:, None] * V4)

    H1 = dout[:, None] * U1
    H2 = din[:, None] * U2
    H3 = (c1 * dout)[:, None] * H3c
    H4 = (c2 * din)[:, None] * H4c
    H5 = (c3 * dout)[:, None] * H5c
    H6 = (c4 * din)[:, None] * H6c

    hcat = jnp.concatenate([H1, H2, H3, H4, H5, H6], axis=1)
    hcat = jnp.pad(hcat, ((0, NPAD - N), (0, 0)))
    a, b, g = alpha, beta, gama
    wcat = jnp.concatenate([
        a * W_sd.T, (1.0 - a) * W_ds.T,
        b * W0.T, (1.0 - b) * W1.T,
        g * W2.T, (1.0 - g) * W3.T,
    ], axis=0)
    bias = (a * b_sd + (1.0 - a) * b_ds + b * b0 + (1.0 - b) * b1
            + g * b2 + (1.0 - g) * b3)[None, :]

    return _tc_combine(hcat, wcat, bias)[:N]
